# baseline (speedup 1.0000x reference)
import functools

import jax
import jax.numpy as jnp
from jax import lax
from jax.experimental import pallas as pl
from jax.experimental.pallas import tpu as pltpu
from jax.experimental.pallas import tpu_sc as plsc

_C = 80
_K = 160
_S = 4
_Q = 4


@functools.lru_cache(maxsize=None)
def _build_sc_call(N, D, T_ROWS):
    info = plsc.get_sparse_core_info()
    num_workers = info.num_cores * info.num_subcores
    KC = _K // _C
    per_w = N // num_workers
    idx_rows = per_w // _C
    nb = per_w // _K
    assert N % num_workers == 0 and per_w % _K == 0 and nb % _S == 0 and nb > 2 * _S

    mesh = plsc.VectorSubcoreMesh(core_axis_name="c", subcore_axis_name="s")

    @functools.partial(
        pl.kernel,
        mesh=mesh,
        compiler_params=pltpu.CompilerParams(use_tc_tiling_on_sc=False),
        out_type=jax.ShapeDtypeStruct((N, D), jnp.float32),
        scratch_types=(
            [pltpu.VMEM((idx_rows, _C), jnp.int32) for _ in range(3)]
            + [pltpu.VMEM((_K, D), jnp.float32) for _ in range(_S)]
            + [pltpu.VMEM_SHARED((T_ROWS, D), jnp.float32)]
            + [pltpu.SemaphoreType.DMA for _ in range(3 * _S + 1)]
        ),
    )
    def sc_fn(t_hbm, l_hbm, u_hbm, et_hbm, el_hbm, eu_hbm, out_hbm, *refs):
        it_v, il_v, iu_v = refs[0:3]
        acc = refs[3:3 + _S]
        shared_t = refs[3 + _S]
        tsem = refs[4 + _S:4 + 2 * _S]
        asem = refs[4 + 2 * _S:4 + 3 * _S]
        osem = refs[4 + 3 * _S:4 + 4 * _S]
        isem = refs[4 + 4 * _S]

        wid = lax.axis_index("s") * info.num_cores + lax.axis_index("c")
        idx_row0 = wid * idx_rows
        out_row0 = wid * per_w

        @pl.when(lax.axis_index("s") == 0)
        def _load_t_table():
            pltpu.sync_copy(et_hbm, shared_t)

        ci = pltpu.async_copy(t_hbm.at[pl.ds(idx_row0, idx_rows)], it_v, isem)
        cl = pltpu.async_copy(l_hbm.at[pl.ds(idx_row0, idx_rows)], il_v, isem)
        cu = pltpu.async_copy(u_hbm.at[pl.ds(idx_row0, idx_rows)], iu_v, isem)
        ci.wait()
        cl.wait()
        cu.wait()
        plsc.subcore_barrier()

        def t_gather(bi, s, issue):
            for k in range(KC):
                d = pltpu.make_async_copy(
                    shared_t.at[it_v.at[bi * KC + k]], acc[s].at[pl.ds(k * _C, _C)], tsem[s])
                d.start() if issue else d.wait()

        def add_gathers(bi, s, issue):
            for k in range(KC):
                dst = acc[s].at[pl.ds(k * _C, _C)]
                dl = pltpu.make_async_copy(el_hbm.at[il_v.at[bi * KC + k]], dst, asem[s])
                du = pltpu.make_async_copy(eu_hbm.at[iu_v.at[bi * KC + k]], dst, asem[s])
                if issue:
                    dl.start(add=True)
                    du.start(add=True)
                else:
                    dl.wait()
                    du.wait()

        def out_copy(bi, s, issue):
            d = pltpu.make_async_copy(acc[s], out_hbm.at[pl.ds(out_row0 + bi * _K, _K)], osem[s])
            d.start() if issue else d.wait()

        def slot_b(i, s):
            t_gather(i - 1, s, False)
            add_gathers(i - 1, s, True)

        def slot_c(i, s):
            add_gathers(i - 2, s, False)
            out_copy(i - 2, s, True)

        for i in range(_S):
            t_gather(i, i % _S, True)
            if i >= 1:
                slot_b(i, (i - 1) % _S)
            if i >= 2:
                slot_c(i, (i - 2) % _S)

        def group(g, carry):
            for s in range(_S):
                i = g * _S + s
                out_copy(i - _S, s, False)
                t_gather(i, s, True)
                slot_b(i, (s - 1) % _S)
                slot_c(i, (s - 2) % _S)
            return carry

        lax.fori_loop(1, nb // _S, group, 0)

        slot_b(nb, (nb - 1) % _S)
        slot_c(nb, (nb - 2) % _S)
        slot_c(nb + 1, (nb - 1) % _S)
        for s in range(_S):
            out_copy(nb - _S + s, s, False)

    return sc_fn


def kernel(t, l, u, emb_t, emb_l, emb_u):
    B, S = t.shape
    D = emb_t.shape[1]
    sq = S // _Q
    Nq = B * sq
    call = _build_sc_call(Nq, D, emb_t.shape[0])
    outs = []
    for q in range(_Q):
        sl = slice(q * sq, (q + 1) * sq)
        ti = t[:, sl].reshape(Nq // _C, _C).astype(jnp.int32)
        li = l[:, sl].reshape(Nq // _C, _C).astype(jnp.int32)
        ui = u[:, sl].reshape(Nq // _C, _C).astype(jnp.int32)
        outs.append(call(ti, li, ui, emb_t, emb_l, emb_u).reshape(B, sq, D))
    return jnp.concatenate(outs, axis=1)


# speedup vs baseline: 1.5585x; 1.5585x over previous
import functools

import jax
import jax.numpy as jnp
from jax import lax
from jax.experimental import pallas as pl
from jax.experimental.pallas import tpu as pltpu
from jax.experimental.pallas import tpu_sc as plsc

_C = 128
_K = 256
_S = 4


@functools.lru_cache(maxsize=None)
def _build_sc_call(N, D, T_ROWS):
    info = plsc.get_sparse_core_info()
    num_workers = info.num_cores * info.num_subcores
    KC = _K // _C
    per_w = N // num_workers
    idx_rows = per_w // _C
    nb = per_w // _K
    assert N % num_workers == 0 and per_w % _K == 0 and nb % _S == 0 and nb > 2 * _S

    mesh = plsc.VectorSubcoreMesh(core_axis_name="c", subcore_axis_name="s")

    @functools.partial(
        pl.kernel,
        mesh=mesh,
        compiler_params=pltpu.CompilerParams(use_tc_tiling_on_sc=False),
        out_type=jax.ShapeDtypeStruct((N, D), jnp.float32),
        scratch_types=(
            [pltpu.VMEM((idx_rows, _C), jnp.int32) for _ in range(3)]
            + [pltpu.VMEM((_K, D), jnp.float32) for _ in range(_S)]
            + [pltpu.VMEM_SHARED((T_ROWS, D), jnp.float32)]
            + [pltpu.SemaphoreType.DMA for _ in range(3 * _S + 1)]
        ),
    )
    def sc_fn(t_hbm, l_hbm, u_hbm, et_hbm, el_hbm, eu_hbm, out_hbm, *refs):
        it_v, il_v, iu_v = refs[0:3]
        acc = refs[3:3 + _S]
        shared_t = refs[3 + _S]
        tsem = refs[4 + _S:4 + 2 * _S]
        asem = refs[4 + 2 * _S:4 + 3 * _S]
        osem = refs[4 + 3 * _S:4 + 4 * _S]
        isem = refs[4 + 4 * _S]

        wid = lax.axis_index("s") * info.num_cores + lax.axis_index("c")
        idx_row0 = wid * idx_rows
        out_row0 = wid * per_w

        @pl.when(lax.axis_index("s") == 0)
        def _load_t_table():
            pltpu.sync_copy(et_hbm, shared_t)

        ci = pltpu.async_copy(t_hbm.at[pl.ds(idx_row0, idx_rows)], it_v, isem)
        cl = pltpu.async_copy(l_hbm.at[pl.ds(idx_row0, idx_rows)], il_v, isem)
        cu = pltpu.async_copy(u_hbm.at[pl.ds(idx_row0, idx_rows)], iu_v, isem)
        ci.wait()
        cl.wait()
        cu.wait()
        plsc.subcore_barrier()

        def t_gather(bi, s, issue):
            for k in range(KC):
                d = pltpu.make_async_copy(
                    shared_t.at[it_v.at[bi * KC + k]], acc[s].at[pl.ds(k * _C, _C)], tsem[s])
                d.start() if issue else d.wait()

        def add_gathers(bi, s, issue):
            for k in range(KC):
                dst = acc[s].at[pl.ds(k * _C, _C)]
                dl = pltpu.make_async_copy(el_hbm.at[il_v.at[bi * KC + k]], dst, asem[s])
                du = pltpu.make_async_copy(eu_hbm.at[iu_v.at[bi * KC + k]], dst, asem[s])
                if issue:
                    dl.start(add=True)
                    du.start(add=True)
                else:
                    dl.wait()
                    du.wait()

        def out_copy(bi, s, issue):
            d = pltpu.make_async_copy(acc[s], out_hbm.at[pl.ds(out_row0 + bi * _K, _K)], osem[s])
            d.start() if issue else d.wait()

        def slot_b(i, s):
            t_gather(i - 1, s, False)
            add_gathers(i - 1, s, True)

        def slot_c(i, s):
            add_gathers(i - 2, s, False)
            out_copy(i - 2, s, True)

        for i in range(_S):
            t_gather(i, i % _S, True)
            if i >= 1:
                slot_b(i, (i - 1) % _S)
            if i >= 2:
                slot_c(i, (i - 2) % _S)

        def group(g, carry):
            for s in range(_S):
                i = g * _S + s
                out_copy(i - _S, s, False)
                t_gather(i, s, True)
                slot_b(i, (s - 1) % _S)
                slot_c(i, (s - 2) % _S)
            return carry

        lax.fori_loop(1, nb // _S, group, 0)

        slot_b(nb, (nb - 1) % _S)
        slot_c(nb, (nb - 2) % _S)
        slot_c(nb + 1, (nb - 1) % _S)
        for s in range(_S):
            out_copy(nb - _S + s, s, False)

    return sc_fn


def kernel(t, l, u, emb_t, emb_l, emb_u):
    B, S = t.shape
    N = B * S
    D = emb_t.shape[1]
    ti = t.reshape(N // _C, _C).astype(jnp.int32)
    li = l.reshape(N // _C, _C).astype(jnp.int32)
    ui = u.reshape(N // _C, _C).astype(jnp.int32)
    out = _build_sc_call(N, D, emb_t.shape[0])(ti, li, ui, emb_t, emb_l, emb_u)
    return out.reshape(B, S, D)


# speedup vs baseline: 2.0778x; 1.3332x over previous
import functools

import jax
import jax.numpy as jnp
from jax import lax
from jax.experimental import pallas as pl
from jax.experimental.pallas import tpu as pltpu
from jax.experimental.pallas import tpu_sc as plsc

_C = 128
_K = 256
_S = 4


@functools.lru_cache(maxsize=None)
def _build_sc_call(N, D, T_ROWS):
    info = plsc.get_sparse_core_info()
    num_workers = info.num_cores * info.num_subcores
    KC = _K // _C
    per_w = N // num_workers
    idx_rows = per_w // _C
    nb = per_w // _K
    assert N % num_workers == 0 and per_w % _K == 0 and nb % _S == 0 and nb > 2 * _S

    mesh = plsc.VectorSubcoreMesh(core_axis_name="c", subcore_axis_name="s")

    @functools.partial(
        pl.kernel,
        mesh=mesh,
        compiler_params=pltpu.CompilerParams(use_tc_tiling_on_sc=False),
        out_type=jax.ShapeDtypeStruct((N, 128), jnp.float32),
        scratch_types=(
            [pltpu.VMEM((idx_rows, _C), jnp.int32) for _ in range(3)]
            + [pltpu.VMEM((_K, D), jnp.float32) for _ in range(_S)]
            + [pltpu.VMEM_SHARED((T_ROWS, D), jnp.float32)]
            + [pltpu.SemaphoreType.DMA for _ in range(3 * _S + 1)]
        ),
    )
    def sc_fn(t_hbm, l_hbm, u_hbm, et_hbm, el_hbm, eu_hbm, out_hbm, *refs):
        it_v, il_v, iu_v = refs[0:3]
        acc = refs[3:3 + _S]
        shared_t = refs[3 + _S]
        tsem = refs[4 + _S:4 + 2 * _S]
        asem = refs[4 + 2 * _S:4 + 3 * _S]
        osem = refs[4 + 3 * _S:4 + 4 * _S]
        isem = refs[4 + 4 * _S]

        wid = lax.axis_index("s") * info.num_cores + lax.axis_index("c")
        idx_row0 = wid * idx_rows
        out_row0 = wid * per_w

        @pl.when(lax.axis_index("s") == 0)
        def _load_t_table():
            pltpu.sync_copy(et_hbm, shared_t)

        ci = pltpu.async_copy(t_hbm.at[pl.ds(idx_row0, idx_rows)], it_v, isem)
        cl = pltpu.async_copy(l_hbm.at[pl.ds(idx_row0, idx_rows)], il_v, isem)
        cu = pltpu.async_copy(u_hbm.at[pl.ds(idx_row0, idx_rows)], iu_v, isem)
        ci.wait()
        cl.wait()
        cu.wait()
        plsc.subcore_barrier()

        def t_gather(bi, s, issue):
            for k in range(KC):
                d = pltpu.make_async_copy(
                    shared_t.at[it_v.at[bi * KC + k]], acc[s].at[pl.ds(k * _C, _C)], tsem[s])
                d.start() if issue else d.wait()

        def add_gathers(bi, s, issue):
            for k in range(KC):
                dst = acc[s].at[pl.ds(k * _C, _C)]
                dl = pltpu.make_async_copy(el_hbm.at[il_v.at[bi * KC + k]], dst, asem[s])
                du = pltpu.make_async_copy(eu_hbm.at[iu_v.at[bi * KC + k]], dst, asem[s])
                if issue:
                    dl.start(add=True)
                    du.start(add=True)
                else:
                    dl.wait()
                    du.wait()

        def out_copy(bi, s, issue):
            d = pltpu.make_async_copy(
                acc[s], out_hbm.at[pl.ds(out_row0 + bi * _K, _K), pl.ds(0, D)], osem[s])
            d.start() if issue else d.wait()

        def slot_b(i, s):
            t_gather(i - 1, s, False)
            add_gathers(i - 1, s, True)

        def slot_c(i, s):
            add_gathers(i - 2, s, False)
            out_copy(i - 2, s, True)

        for i in range(_S):
            t_gather(i, i % _S, True)
            if i >= 1:
                slot_b(i, (i - 1) % _S)
            if i >= 2:
                slot_c(i, (i - 2) % _S)

        def group(g, carry):
            for s in range(_S):
                i = g * _S + s
                out_copy(i - _S, s, False)
                t_gather(i, s, True)
                slot_b(i, (s - 1) % _S)
                slot_c(i, (s - 2) % _S)
            return carry

        lax.fori_loop(1, nb // _S, group, 0)

        slot_b(nb, (nb - 1) % _S)
        slot_c(nb, (nb - 2) % _S)
        slot_c(nb + 1, (nb - 1) % _S)
        for s in range(_S):
            out_copy(nb - _S + s, s, False)

    return sc_fn


def kernel(t, l, u, emb_t, emb_l, emb_u):
    B, S = t.shape
    N = B * S
    D = emb_t.shape[1]
    ti = t.reshape(N // _C, _C).astype(jnp.int32)
    li = l.reshape(N // _C, _C).astype(jnp.int32)
    ui = u.reshape(N // _C, _C).astype(jnp.int32)
    out = _build_sc_call(N, D, emb_t.shape[0])(ti, li, ui, emb_t, emb_l, emb_u)
    return out[:, :D].reshape(B, S, D)
